# baseline (speedup 1.0000x reference)
import functools

import jax
import jax.numpy as jnp
from jax import lax
from jax.experimental import pallas as pl
from jax.experimental.pallas import tpu as pltpu
from jax.experimental.pallas import tpu_sc as plsc

_BLOCK_N = 2000

_NC = 2
_NS = 16
_NW = _NC * _NS
_L = 16
_CHUNK = 16000


def _ew_kernel(theta_mu_ref, log_theta_s_ref, eps_theta_ref,
               z_w_ref, z_b_ref, log_z_s_ref, eps_z_ref, out_ref):
    theta = theta_mu_ref[:] + jnp.exp(log_theta_s_ref[:]) * eps_theta_ref[:]
    out_ref[:] = (z_w_ref[:] * theta + z_b_ref[:]
                  + jnp.exp(log_z_s_ref[:]) * eps_z_ref[:])


def _tc_kernel(theta_mu, log_theta_s, z_w, z_b, log_z_s, eps_theta, eps_z):
    n, j = z_w.shape
    block_n = _BLOCK_N if n % _BLOCK_N == 0 else n
    grid = (n // block_n,)

    small = pl.BlockSpec((1, j), lambda i: (0, 0))
    big = pl.BlockSpec((block_n, j), lambda i: (i, 0))

    return pl.pallas_call(
        _ew_kernel,
        grid=grid,
        in_specs=[small, small, small, big, big, big, big],
        out_specs=big,
        out_shape=jax.ShapeDtypeStruct((n, j), z_w.dtype),
    )(theta_mu.reshape(1, j), log_theta_s.reshape(1, j),
      eps_theta.reshape(1, j), z_w, z_b, log_z_s, eps_z)


def _make_sc_kernel(total_words, j, chunk):
    w_per_worker = total_words // _NW
    n_chunks = w_per_worker // chunk
    rows_per_chunk = chunk // j
    j_vregs = j // _L
    mesh = plsc.VectorSubcoreMesh(core_axis_name="c", subcore_axis_name="s")

    @functools.partial(
        pl.kernel,
        mesh=mesh,
        out_type=jax.ShapeDtypeStruct((total_words,), jnp.float32),
        scratch_types=[
            pltpu.VMEM((j,), jnp.float32),
            pltpu.VMEM((j,), jnp.float32),
            pltpu.VMEM((j,), jnp.float32),
            pltpu.VMEM((chunk,), jnp.float32),
            pltpu.VMEM((chunk,), jnp.float32),
            pltpu.VMEM((chunk,), jnp.float32),
            pltpu.VMEM((chunk,), jnp.float32),
        ],
    )
    def sc_k(tmu_hbm, lts_hbm, ept_hbm, zw_hbm, zb_hbm, ls_hbm, ez_hbm,
             out_hbm, tmu_v, lts_v, ept_v, zw_v, zb_v, ls_v, ez_v):
        wid = lax.axis_index("s") * _NC + lax.axis_index("c")
        base = wid * w_per_worker

        pltpu.sync_copy(tmu_hbm, tmu_v)
        pltpu.sync_copy(lts_hbm, lts_v)
        pltpu.sync_copy(ept_hbm, ept_v)
        thetas = [
            tmu_v[pl.ds(v * _L, _L)]
            + jnp.exp(lts_v[pl.ds(v * _L, _L)]) * ept_v[pl.ds(v * _L, _L)]
            for v in range(j_vregs)
        ]

        def chunk_body(k, _):
            off = base + k * chunk
            pltpu.sync_copy(zw_hbm.at[pl.ds(off, chunk)], zw_v)
            pltpu.sync_copy(zb_hbm.at[pl.ds(off, chunk)], zb_v)
            pltpu.sync_copy(ls_hbm.at[pl.ds(off, chunk)], ls_v)
            pltpu.sync_copy(ez_hbm.at[pl.ds(off, chunk)], ez_v)

            def row_body(r, _):
                rb = r * j
                for v in range(j_vregs):
                    sl = pl.ds(rb + v * _L, _L)
                    ez_v[sl] = (zw_v[sl] * thetas[v] + zb_v[sl]
                                + jnp.exp(ls_v[sl]) * ez_v[sl])
                return 0

            lax.fori_loop(0, rows_per_chunk, row_body, 0)
            pltpu.sync_copy(ez_v, out_hbm.at[pl.ds(off, chunk)])
            return 0

        lax.fori_loop(0, n_chunks, chunk_body, 0)

    return sc_k


_SC_CHUNK = 16000
_N_SC = 20000
_TC_BLOCK = 4000


def _tc_tail(theta_mu, log_theta_s, eps_theta, z_w, z_b, log_z_s, eps_z,
             n_sc, block_n):
    n, j = z_w.shape
    n_tc = n - n_sc
    grid = (n_tc // block_n,)
    off = n_sc // block_n

    small = pl.BlockSpec((1, j), lambda i: (0, 0))
    big = pl.BlockSpec((block_n, j), lambda i: (i + off, 0))

    return pl.pallas_call(
        _ew_kernel,
        grid=grid,
        in_specs=[small, small, small, big, big, big, big],
        out_specs=big,
        out_shape=jax.ShapeDtypeStruct((n, j), z_w.dtype),
    )(theta_mu.reshape(1, j), log_theta_s.reshape(1, j),
      eps_theta.reshape(1, j), z_w, z_b, log_z_s, eps_z)


def kernel(theta_mu, log_theta_s, z_w, z_b, log_z_s, eps_theta, eps_z):
    return _tc_kernel(theta_mu, log_theta_s, z_w, z_b, log_z_s,
                      eps_theta, eps_z)
    n, j = z_w.shape
    n_sc = _N_SC
    sc_words = n_sc * j
    hybrid_ok = (
        n_sc < n
        and j % _L == 0
        and sc_words % (_NW * _SC_CHUNK) == 0
        and _SC_CHUNK % j == 0
        and (n - n_sc) % _TC_BLOCK == 0
        and n_sc % _TC_BLOCK == 0
    )
    if not hybrid_ok:
        return _tc_kernel(theta_mu, log_theta_s, z_w, z_b, log_z_s,
                          eps_theta, eps_z)

    sc_k = _make_sc_kernel(sc_words, j, _SC_CHUNK)
    sc_flat = sc_k(theta_mu, log_theta_s, eps_theta,
                   z_w.reshape(-1), z_b.reshape(-1),
                   log_z_s.reshape(-1), eps_z.reshape(-1))
    tc_full = _tc_tail(theta_mu, log_theta_s, eps_theta,
                       z_w, z_b, log_z_s, eps_z, n_sc, _TC_BLOCK)
    return jax.lax.dynamic_update_slice(
        tc_full, sc_flat.reshape(n_sc, j), (0, 0))


# speedup vs baseline: 1.0368x; 1.0368x over previous
import functools

import jax
import jax.numpy as jnp
from jax import lax
from jax.experimental import pallas as pl
from jax.experimental.pallas import tpu as pltpu
from jax.experimental.pallas import tpu_sc as plsc

_BLOCK_N = 5000

_NC = 2
_NS = 16
_NW = _NC * _NS
_L = 16
_CHUNK = 16000


def _ew_kernel(theta_mu_ref, log_theta_s_ref, eps_theta_ref,
               z_w_ref, z_b_ref, log_z_s_ref, eps_z_ref, out_ref):
    theta = theta_mu_ref[:] + jnp.exp(log_theta_s_ref[:]) * eps_theta_ref[:]
    out_ref[:] = (z_w_ref[:] * theta + z_b_ref[:]
                  + jnp.exp(log_z_s_ref[:]) * eps_z_ref[:])


def _tc_kernel(theta_mu, log_theta_s, z_w, z_b, log_z_s, eps_theta, eps_z):
    n, j = z_w.shape
    block_n = _BLOCK_N if n % _BLOCK_N == 0 else n
    grid = (n // block_n,)

    small = pl.BlockSpec((1, j), lambda i: (0, 0))
    big = pl.BlockSpec((block_n, j), lambda i: (i, 0))

    return pl.pallas_call(
        _ew_kernel,
        grid=grid,
        in_specs=[small, small, small, big, big, big, big],
        out_specs=big,
        out_shape=jax.ShapeDtypeStruct((n, j), z_w.dtype),
    )(theta_mu.reshape(1, j), log_theta_s.reshape(1, j),
      eps_theta.reshape(1, j), z_w, z_b, log_z_s, eps_z)


def _make_sc_kernel(total_words, j, chunk):
    w_per_worker = total_words // _NW
    n_chunks = w_per_worker // chunk
    rows_per_chunk = chunk // j
    j_vregs = j // _L
    mesh = plsc.VectorSubcoreMesh(core_axis_name="c", subcore_axis_name="s")

    @functools.partial(
        pl.kernel,
        mesh=mesh,
        out_type=jax.ShapeDtypeStruct((total_words,), jnp.float32),
        scratch_types=[
            pltpu.VMEM((j,), jnp.float32),
            pltpu.VMEM((j,), jnp.float32),
            pltpu.VMEM((j,), jnp.float32),
            pltpu.VMEM((chunk,), jnp.float32),
            pltpu.VMEM((chunk,), jnp.float32),
            pltpu.VMEM((chunk,), jnp.float32),
            pltpu.VMEM((chunk,), jnp.float32),
        ],
    )
    def sc_k(tmu_hbm, lts_hbm, ept_hbm, zw_hbm, zb_hbm, ls_hbm, ez_hbm,
             out_hbm, tmu_v, lts_v, ept_v, zw_v, zb_v, ls_v, ez_v):
        wid = lax.axis_index("s") * _NC + lax.axis_index("c")
        base = wid * w_per_worker

        pltpu.sync_copy(tmu_hbm, tmu_v)
        pltpu.sync_copy(lts_hbm, lts_v)
        pltpu.sync_copy(ept_hbm, ept_v)
        thetas = [
            tmu_v[pl.ds(v * _L, _L)]
            + jnp.exp(lts_v[pl.ds(v * _L, _L)]) * ept_v[pl.ds(v * _L, _L)]
            for v in range(j_vregs)
        ]

        def chunk_body(k, _):
            off = base + k * chunk
            pltpu.sync_copy(zw_hbm.at[pl.ds(off, chunk)], zw_v)
            pltpu.sync_copy(zb_hbm.at[pl.ds(off, chunk)], zb_v)
            pltpu.sync_copy(ls_hbm.at[pl.ds(off, chunk)], ls_v)
            pltpu.sync_copy(ez_hbm.at[pl.ds(off, chunk)], ez_v)

            def row_body(r, _):
                rb = r * j
                for v in range(j_vregs):
                    sl = pl.ds(rb + v * _L, _L)
                    ez_v[sl] = (zw_v[sl] * thetas[v] + zb_v[sl]
                                + jnp.exp(ls_v[sl]) * ez_v[sl])
                return 0

            lax.fori_loop(0, rows_per_chunk, row_body, 0)
            pltpu.sync_copy(ez_v, out_hbm.at[pl.ds(off, chunk)])
            return 0

        lax.fori_loop(0, n_chunks, chunk_body, 0)

    return sc_k


_SC_CHUNK = 16000
_N_SC = 20000
_TC_BLOCK = 4000


def _tc_tail(theta_mu, log_theta_s, eps_theta, z_w, z_b, log_z_s, eps_z,
             n_sc, block_n):
    n, j = z_w.shape
    n_tc = n - n_sc
    grid = (n_tc // block_n,)
    off = n_sc // block_n

    small = pl.BlockSpec((1, j), lambda i: (0, 0))
    big = pl.BlockSpec((block_n, j), lambda i: (i + off, 0))

    return pl.pallas_call(
        _ew_kernel,
        grid=grid,
        in_specs=[small, small, small, big, big, big, big],
        out_specs=big,
        out_shape=jax.ShapeDtypeStruct((n, j), z_w.dtype),
    )(theta_mu.reshape(1, j), log_theta_s.reshape(1, j),
      eps_theta.reshape(1, j), z_w, z_b, log_z_s, eps_z)


def kernel(theta_mu, log_theta_s, z_w, z_b, log_z_s, eps_theta, eps_z):
    return _tc_kernel(theta_mu, log_theta_s, z_w, z_b, log_z_s,
                      eps_theta, eps_z)
    n, j = z_w.shape
    n_sc = _N_SC
    sc_words = n_sc * j
    hybrid_ok = (
        n_sc < n
        and j % _L == 0
        and sc_words % (_NW * _SC_CHUNK) == 0
        and _SC_CHUNK % j == 0
        and (n - n_sc) % _TC_BLOCK == 0
        and n_sc % _TC_BLOCK == 0
    )
    if not hybrid_ok:
        return _tc_kernel(theta_mu, log_theta_s, z_w, z_b, log_z_s,
                          eps_theta, eps_z)

    sc_k = _make_sc_kernel(sc_words, j, _SC_CHUNK)
    sc_flat = sc_k(theta_mu, log_theta_s, eps_theta,
                   z_w.reshape(-1), z_b.reshape(-1),
                   log_z_s.reshape(-1), eps_z.reshape(-1))
    tc_full = _tc_tail(theta_mu, log_theta_s, eps_theta,
                       z_w, z_b, log_z_s, eps_z, n_sc, _TC_BLOCK)
    return jax.lax.dynamic_update_slice(
        tc_full, sc_flat.reshape(n_sc, j), (0, 0))
